# baseline (speedup 1.0000x reference)
import functools

import jax
import jax.numpy as jnp
from jax import lax
from jax.experimental import pallas as pl
from jax.experimental.pallas import tpu as pltpu
from jax.experimental.pallas import tpu_sc as plsc

B, N, T, D = 32, 500, 12, 128
_NN = 5
_NCH = N // _NN
_L = 16


def _sc_body(x_hbm, e_hbm, o_hbm, xin, xout, ebuf, i0, i1, o0, o1, esem):
    info = plsc.get_sparse_core_info()
    nc = info.num_cores
    w = lax.axis_index("s") * nc + lax.axis_index("c")
    isems = (i0, i1)
    osems = (o0, o1)

    def in_copy(c, s):
        return pltpu.make_async_copy(
            x_hbm.at[w, pl.ds(c * _NN, _NN)], xin.at[s], isems[s])

    def out_copy(c, s):
        return pltpu.make_async_copy(
            xout.at[s], o_hbm.at[w, pl.ds(c * _NN, _NN)], osems[s])

    pltpu.make_async_copy(e_hbm, ebuf, esem).start()
    in_copy(0, 0).start()
    in_copy(1, 1).start()
    pltpu.make_async_copy(e_hbm, ebuf, esem).wait()

    def compute_chunk(c, s):
        xin_s, xout_s = xin.at[s], xout.at[s]

        def n_body(n, _):
            ng = c * _NN + n
            ev = [ebuf[ng, pl.ds(l * _L, _L)] for l in range(D // _L)]
            for t in range(T):
                for l in range(D // _L):
                    xout_s[n, t, pl.ds(l * _L, _L)] = (
                        xin_s[n, t, pl.ds(l * _L, _L)] + ev[l])
            return 0

        lax.fori_loop(0, _NN, n_body, 0)

    def round_body(r, _):
        for s in range(2):
            c = 2 * r + s
            in_copy(c, s).wait()

            @pl.when(r >= 1)
            def _():
                out_copy(c - 2, s).wait()

            compute_chunk(c, s)
            out_copy(c, s).start()

            @pl.when(r < _NCH // 2 - 1)
            def _():
                in_copy(c + 2, s).start()

        return 0

    lax.fori_loop(0, _NCH // 2, round_body, 0)
    out_copy(_NCH - 2, 0).wait()
    out_copy(_NCH - 1, 1).wait()


@functools.partial(
    pl.kernel,
    out_type=jax.ShapeDtypeStruct((B, N, T, D), jnp.float32),
    mesh=plsc.VectorSubcoreMesh(core_axis_name="c", subcore_axis_name="s"),
    scratch_types=[
        pltpu.VMEM((2, _NN, T, D), jnp.float32),
        pltpu.VMEM((2, _NN, T, D), jnp.float32),
        pltpu.VMEM((N, D), jnp.float32),
        pltpu.SemaphoreType.DMA,
        pltpu.SemaphoreType.DMA,
        pltpu.SemaphoreType.DMA,
        pltpu.SemaphoreType.DMA,
        pltpu.SemaphoreType.DMA,
    ],
)
def _sc_kernel(x_hbm, e_hbm, o_hbm, xin, xout, ebuf, i0, i1, o0, o1, esem):
    _sc_body(x_hbm, e_hbm, o_hbm, xin, xout, ebuf, i0, i1, o0, o1, esem)


def kernel(x, emb_weight):
    return _sc_kernel(x, emb_weight)


# speedup vs baseline: 4.2403x; 4.2403x over previous
import jax
import jax.numpy as jnp
from jax.experimental import pallas as pl
from jax.experimental.pallas import tpu as pltpu

_NB = 32


def _add_kernel(x_ref, e_ref, o_ref):
    o_ref[...] = x_ref[...] + e_ref[...][:, None, None, :]


def kernel(x, emb_weight):
    B, N, T, D = x.shape
    xt = jnp.transpose(x, (1, 2, 0, 3))
    out = pl.pallas_call(
        _add_kernel,
        grid=(pl.cdiv(N, _NB),),
        in_specs=[
            pl.BlockSpec((_NB, T, B, D), lambda j: (j, 0, 0, 0)),
            pl.BlockSpec((_NB, D), lambda j: (j, 0)),
        ],
        out_specs=pl.BlockSpec((_NB, T, B, D), lambda j: (j, 0, 0, 0)),
        out_shape=jax.ShapeDtypeStruct((N, T, B, D), x.dtype),
        compiler_params=pltpu.CompilerParams(
            dimension_semantics=("parallel",),
        ),
    )(xt, emb_weight)
    return jnp.transpose(out, (2, 0, 1, 3))


# speedup vs baseline: 4.2959x; 1.0131x over previous
import jax
import jax.numpy as jnp
from jax.experimental import pallas as pl
from jax.experimental.pallas import tpu as pltpu

_NB = 64


def _add_kernel(x_ref, e_ref, o_ref):
    o_ref[...] = x_ref[...] + e_ref[...][:, None, None, :]


def kernel(x, emb_weight):
    B, N, T, D = x.shape
    xt = jnp.transpose(x, (1, 2, 0, 3))
    out = pl.pallas_call(
        _add_kernel,
        grid=(pl.cdiv(N, _NB),),
        in_specs=[
            pl.BlockSpec((_NB, T, B, D), lambda j: (j, 0, 0, 0)),
            pl.BlockSpec((_NB, D), lambda j: (j, 0)),
        ],
        out_specs=pl.BlockSpec((_NB, T, B, D), lambda j: (j, 0, 0, 0)),
        out_shape=jax.ShapeDtypeStruct((N, T, B, D), x.dtype),
        compiler_params=pltpu.CompilerParams(
            dimension_semantics=("parallel",),
        ),
    )(xt, emb_weight)
    return jnp.transpose(out, (2, 0, 1, 3))
